# baseline (speedup 1.0000x reference)
import jax
import jax.numpy as jnp
from jax.experimental import pallas as pl
from jax.experimental.pallas import tpu as pltpu


def _mlp3_kernel(x_ref, w1_ref, b1_ref, w2_ref, b2_ref, w3_ref, b3_ref,
                 o_ref, w1b, w2b, xb, h1b, h2b):
    n = b1_ref.shape[1]
    tnc = w1_ref.shape[1]
    nc = n // tnc
    nrows = pl.num_programs(0) - 2 * nc + 1
    s = pl.program_id(0)

    def layer1_wide(h1_dst):
        xr = x_ref[...].astype(jnp.bfloat16)
        for c in range(nc):
            sl = pl.ds(c * tnc, tnc)
            acc = jnp.dot(xr, w1b[:, sl], preferred_element_type=jnp.float32)
            h1_dst[:, sl] = jnp.maximum(
                acc + b1_ref[:, sl], 0.0).astype(jnp.bfloat16)

    def layer23_wide(h1_src):
        for c in range(nc):
            sl = pl.ds(c * tnc, tnc)
            acc = jnp.dot(h1_src[...], w2b[:, sl],
                          preferred_element_type=jnp.float32)
            h2b[:, sl] = jnp.maximum(
                acc + b2_ref[:, sl], 0.0).astype(jnp.bfloat16)
        w3c = w3_ref[...].astype(jnp.bfloat16)
        y = jnp.dot(h2b[...], w3c, preferred_element_type=jnp.float32)
        o_ref[...] = y + b3_ref[...]

    @pl.when(s == 0)
    def _cast_x0():
        xb[...] = x_ref[...].astype(jnp.bfloat16)

    @pl.when(s < nc)
    def _stream_w1_chunk():
        sl = pl.ds(s * tnc, tnc)
        wc = w1_ref[...].astype(jnp.bfloat16)
        w1b[:, sl] = wc
        acc = jnp.dot(xb[...], wc, preferred_element_type=jnp.float32)
        h1b[pl.ds(0, 1), :, sl] = jnp.maximum(
            acc + b1_ref[:, sl], 0.0).astype(jnp.bfloat16)[None]

    @pl.when((s >= nc) & (s < 2 * nc))
    def _stream_w2_chunk():
        c = s - nc
        sl = pl.ds(c * tnc, tnc)
        wc = w2_ref[...].astype(jnp.bfloat16)
        w2b[:, sl] = wc
        acc = jnp.dot(h1b[0], wc, preferred_element_type=jnp.float32)
        h2b[:, sl] = jnp.maximum(acc + b2_ref[:, sl], 0.0).astype(jnp.bfloat16)

    @pl.when(s == 2 * nc - 1)
    def _row0_out():
        w3c = w3_ref[...].astype(jnp.bfloat16)
        y = jnp.dot(h2b[...], w3c, preferred_element_type=jnp.float32)
        o_ref[...] = y + b3_ref[...]

    t = s - 2 * nc
    r_new = t + 1
    par_new = jax.lax.rem(r_new, jnp.int32(2))
    par_old = 1 - par_new

    @pl.when(t == 0)
    def _pipe_head():
        layer1_wide(h1b.at[par_new])

    @pl.when((t >= 1) & (t < nrows - 1))
    def _pipe_mid():
        layer1_wide(h1b.at[par_new])
        layer23_wide(h1b.at[par_old])

    @pl.when(t == nrows - 1)
    def _pipe_tail():
        layer23_wide(h1b.at[par_old])


def _mlp3(x, w1, b1r, w2, b2r, w3, b3r, *, tm, tnc):
    M, K = x.shape
    N = w1.shape[1]
    O = w3.shape[1]
    nc = N // tnc
    nrows = M // tm
    nsteps = 2 * nc + nrows
    flops = 2 * M * K * N + 2 * M * N * N + 2 * M * N * O
    bytes_accessed = 4 * (M * K + K * N + N * N + N * O + M * O)

    x_row = lambda s: jnp.clip(s - (2 * nc - 1), 0, nrows - 1)
    o_row = lambda s: jnp.clip(s - 2 * nc, 0, nrows - 1)
    return pl.pallas_call(
        _mlp3_kernel,
        out_shape=jax.ShapeDtypeStruct((M, O), jnp.float32),
        grid=(nsteps,),
        in_specs=[
            pl.BlockSpec((tm, K), lambda s: (x_row(s), 0)),
            pl.BlockSpec((K, tnc), lambda s: (0, jnp.minimum(s, nc - 1))),
            pl.BlockSpec((1, N), lambda s: (0, 0)),
            pl.BlockSpec((K, tnc),
                         lambda s: (0, jnp.clip(s - nc, 0, nc - 1))),
            pl.BlockSpec((1, N), lambda s: (0, 0)),
            pl.BlockSpec((N, O), lambda s: (0, 0)),
            pl.BlockSpec((1, O), lambda s: (0, 0)),
        ],
        out_specs=pl.BlockSpec((tm, O), lambda s: (o_row(s), 0)),
        scratch_shapes=[
            pltpu.VMEM((K, N), jnp.bfloat16),
            pltpu.VMEM((N, N), jnp.bfloat16),
            pltpu.VMEM((tm, K), jnp.bfloat16),
            pltpu.VMEM((2, tm, N), jnp.bfloat16),
            pltpu.VMEM((tm, N), jnp.bfloat16),
        ],
        compiler_params=pltpu.CompilerParams(
            dimension_semantics=("arbitrary",),
        ),
        cost_estimate=pl.CostEstimate(
            flops=flops, transcendentals=0, bytes_accessed=bytes_accessed
        ),
    )(x, w1, b1r, w2, b2r, w3, b3r)


@jax.jit
def kernel(x, w1, b1, w2, b2, w3, b3):
    M = x.shape[0]
    N = w1.shape[1]
    O = w3.shape[1]
    tm = min(512, max(M // 4, 8))
    tnc = min(512, max(N // 2, 128))
    return _mlp3(x, w1, b1.reshape(1, N), w2, b2.reshape(1, N),
                 w3, b3.reshape(1, O), tm=tm, tnc=tnc)


# speedup vs baseline: 1.0969x; 1.0969x over previous
import jax
import jax.numpy as jnp
from jax.experimental import pallas as pl
from jax.experimental.pallas import tpu as pltpu

_WIDE = 1024


def _mlp3_kernel(x_ref, w1_ref, b1_ref, w2_ref, b2_ref, w3_ref, b3_ref,
                 o_ref, w1b, w2b, xb, h1b, h2b):
    n = b1_ref.shape[1]
    tnc = w1_ref.shape[1]
    nc = n // tnc
    s = pl.program_id(0)

    def finish(h2full):
        w3c = w3_ref[...].astype(jnp.bfloat16)
        y = jnp.dot(h2full, w3c, preferred_element_type=jnp.float32)
        o_ref[...] = y + b3_ref[...]

    @pl.when(s == 0)
    def _cast_x0():
        xb[...] = x_ref[...].astype(jnp.bfloat16)

    @pl.when(s < nc)
    def _stream_w1_chunk():
        sl = pl.ds(s * tnc, tnc)
        wc = w1_ref[...].astype(jnp.bfloat16)
        w1b[:, sl] = wc
        acc = jnp.dot(xb[...], wc, preferred_element_type=jnp.float32)
        h1b[:, sl] = jnp.maximum(acc + b1_ref[:, sl], 0.0).astype(jnp.bfloat16)

    @pl.when((s >= nc) & (s < 2 * nc))
    def _stream_w2_chunk():
        c = s - nc
        sl = pl.ds(c * tnc, tnc)
        wc = w2_ref[...].astype(jnp.bfloat16)
        w2b[:, sl] = wc
        acc = jnp.dot(h1b[...], wc, preferred_element_type=jnp.float32)
        h2b[:, sl] = jnp.maximum(acc + b2_ref[:, sl], 0.0).astype(jnp.bfloat16)

    @pl.when(s == 2 * nc - 1)
    def _row0_out():
        finish(h2b[...])

    @pl.when(s >= 2 * nc)
    def _later_rows():
        wide = min(_WIDE, n)
        xr = x_ref[...].astype(jnp.bfloat16)
        for c in range(n // wide):
            sl = pl.ds(c * wide, wide)
            acc = jnp.dot(xr, w1b[:, sl], preferred_element_type=jnp.float32)
            h1b[:, sl] = jnp.maximum(
                acc + b1_ref[:, sl], 0.0).astype(jnp.bfloat16)
        for c in range(n // wide):
            sl = pl.ds(c * wide, wide)
            acc = jnp.dot(h1b[...], w2b[:, sl],
                          preferred_element_type=jnp.float32)
            h2b[:, sl] = jnp.maximum(
                acc + b2_ref[:, sl], 0.0).astype(jnp.bfloat16)
        finish(h2b[...])


def _mlp3(x, w1, b1r, w2, b2r, w3, b3r, *, tm, tnc):
    M, K = x.shape
    N = w1.shape[1]
    O = w3.shape[1]
    nc = N // tnc
    nrows = M // tm
    nsteps = 2 * nc + (nrows - 1)
    flops = 2 * M * K * N + 2 * M * N * N + 2 * M * N * O
    bytes_accessed = 4 * (M * K + K * N + N * N + N * O + M * O)

    row_of = lambda s: jnp.maximum(s - (2 * nc - 1), 0)
    return pl.pallas_call(
        _mlp3_kernel,
        out_shape=jax.ShapeDtypeStruct((M, O), jnp.float32),
        grid=(nsteps,),
        in_specs=[
            pl.BlockSpec((tm, K), lambda s: (row_of(s), 0)),
            pl.BlockSpec((K, tnc), lambda s: (0, jnp.minimum(s, nc - 1))),
            pl.BlockSpec((1, N), lambda s: (0, 0)),
            pl.BlockSpec((K, tnc),
                         lambda s: (0, jnp.clip(s - nc, 0, nc - 1))),
            pl.BlockSpec((1, N), lambda s: (0, 0)),
            pl.BlockSpec((N, O), lambda s: (0, 0)),
            pl.BlockSpec((1, O), lambda s: (0, 0)),
        ],
        out_specs=pl.BlockSpec((tm, O), lambda s: (row_of(s), 0)),
        scratch_shapes=[
            pltpu.VMEM((K, N), jnp.bfloat16),
            pltpu.VMEM((N, N), jnp.bfloat16),
            pltpu.VMEM((tm, K), jnp.bfloat16),
            pltpu.VMEM((tm, N), jnp.bfloat16),
            pltpu.VMEM((tm, N), jnp.bfloat16),
        ],
        compiler_params=pltpu.CompilerParams(
            dimension_semantics=("arbitrary",),
        ),
        cost_estimate=pl.CostEstimate(
            flops=flops, transcendentals=0, bytes_accessed=bytes_accessed
        ),
    )(x, w1, b1r, w2, b2r, w3, b3r)


@jax.jit
def kernel(x, w1, b1, w2, b2, w3, b3):
    M = x.shape[0]
    N = w1.shape[1]
    O = w3.shape[1]
    tm = min(512, max(M // 4, 8))
    tnc = min(512, max(N // 2, 128))
    return _mlp3(x, w1, b1.reshape(1, N), w2, b2.reshape(1, N),
                 w3, b3.reshape(1, O), tm=tm, tnc=tnc)
